# baseline (device time: 13214 ns/iter reference)
import jax
import jax.numpy as jnp
from jax import lax
from jax.experimental import pallas as pl
from jax.experimental.pallas import tpu as pltpu

N_DEV = 4


def kernel(x, w_mat):
    k_glob, m_per = x.shape
    _, n = w_mat.shape
    blk = m_per
    half = n // 2

    x = x.astype(jnp.bfloat16)

    def body(x_ref, w_hbm, out_hbm, wv_ref, comm_ref, yv_ref,
             send_sems, recv_sems, wdma_sems, out_sems):
        my = lax.axis_index("i")

        wdmas = []
        for k in range(N_DEV):
            j = [my, (my - 1) % N_DEV, (my + 1) % N_DEV, (my + 2) % N_DEV][k]
            dma = pltpu.make_async_copy(
                w_hbm.at[pl.ds(j * blk, blk), :],
                wv_ref.at[k],
                wdma_sems.at[k],
            )
            dma.start()
            wdmas.append(dma)

        barrier_sem = pltpu.get_barrier_semaphore()
        for k in range(1, N_DEV):
            peer = (my + k) % N_DEV
            pl.semaphore_signal(
                barrier_sem, inc=1,
                device_id=(peer,), device_id_type=pl.DeviceIdType.MESH,
            )
        pl.semaphore_wait(barrier_sem, N_DEV - 1)

        rdmas = []
        for k in range(1, N_DEV):
            peer = (my + k) % N_DEV
            rdma = pltpu.make_async_remote_copy(
                src_ref=x_ref.at[pl.ds(peer * blk, blk), :],
                dst_ref=comm_ref.at[k - 1],
                send_sem=send_sems.at[k - 1],
                recv_sem=recv_sems.at[k - 1],
                device_id=(peer,),
                device_id_type=pl.DeviceIdType.MESH,
            )
            rdma.start()
            rdmas.append(rdma)

        wdmas[0].wait()
        acc = jnp.dot(
            x_ref[pl.ds(my * blk, blk), :],
            wv_ref[0].astype(jnp.bfloat16),
            preferred_element_type=jnp.float32,
        )

        for k, wslot in ((1, 1), (3, 2)):
            rdmas[k - 1].wait()
            wdmas[wslot].wait()
            acc += jnp.dot(
                comm_ref[k - 1],
                wv_ref[wslot].astype(jnp.bfloat16),
                preferred_element_type=jnp.float32,
            )

        rdmas[1].wait()
        wdmas[3].wait()
        out_dmas = []
        for h in range(2):
            cols = pl.ds(h * half, half)
            yh = acc[:, h * half:(h + 1) * half] + jnp.dot(
                comm_ref[1],
                wv_ref[3].astype(jnp.bfloat16)[:, h * half:(h + 1) * half],
                preferred_element_type=jnp.float32,
            )
            yv_ref[:, cols] = yh.astype(jnp.bfloat16)
            dma = pltpu.make_async_copy(
                yv_ref.at[:, cols], out_hbm.at[:, cols], out_sems.at[h]
            )
            dma.start()
            out_dmas.append(dma)
        for dma in out_dmas:
            dma.wait()

    return pl.pallas_call(
        body,
        out_shape=jax.ShapeDtypeStruct((blk, n), jnp.bfloat16),
        in_specs=[
            pl.BlockSpec(memory_space=pltpu.VMEM),
            pl.BlockSpec(memory_space=pl.ANY),
        ],
        out_specs=pl.BlockSpec(memory_space=pl.ANY),
        scratch_shapes=[
            pltpu.VMEM((N_DEV, blk, n), jnp.float32),
            pltpu.VMEM((N_DEV - 1, blk, blk), jnp.bfloat16),
            pltpu.VMEM((blk, n), jnp.bfloat16),
            pltpu.SemaphoreType.DMA((N_DEV - 1,)),
            pltpu.SemaphoreType.DMA((N_DEV - 1,)),
            pltpu.SemaphoreType.DMA((N_DEV,)),
            pltpu.SemaphoreType.DMA((2,)),
        ],
        compiler_params=pltpu.CompilerParams(collective_id=0),
    )(x, w_mat)
